# baseline (speedup 1.0000x reference)
import jax
import jax.numpy as jnp
from jax import lax
from jax.experimental import pallas as pl
from jax.experimental.pallas import tpu as pltpu
from jax.experimental.pallas import tpu_sc as plsc

_N = 4096
_B = 4
_TN = 512
_S_TC = 3072
_NW = 32
_WPB = _NW // _B
_RW = (_N - _S_TC) // _WPB


def _lane_shuffle(v, perm):
    return lax.gather(
        v,
        perm,
        dimension_numbers=lax.GatherDimensionNumbers(
            offset_dims=(), collapsed_slice_dims=(0,), start_index_map=(0,)
        ),
        slice_sizes=(1,),
        mode=lax.GatherScatterMode.PROMISE_IN_BOUNDS,
    )


def _prep_block(x1_ref, x2_ref,
                x1x_ref, x1y_ref, x1z_ref, x2x_ref, x2y_ref, x2z_ref):
    a = x1_ref[0].T
    c = x2_ref[0].T
    x1x_ref[...] = a[0]
    x1y_ref[...] = a[1]
    x1z_ref[...] = a[2]
    x2x_ref[...] = c[0]
    x2y_ref[...] = c[1]
    x2z_ref[...] = c[2]


def _tc_block(x1_ref, x2x_ref, x2y_ref, x2z_ref,
              d1_ref, i1_ref, d2_ref, i2_ref):
    i = pl.program_id(1)
    a = x1_ref[0]
    n = x2x_ref.shape[0]
    bs = (x2x_ref[...], x2y_ref[...], x2z_ref[...])

    d = None
    for k in range(3):
        ak = a[:, k : k + 1]
        bk = bs[k].reshape(1, n)
        t = ak - bk
        t = t * t
        d = t if d is None else d + t

    m1 = jnp.min(d, axis=1, keepdims=True)
    jiota = lax.broadcasted_iota(jnp.int32, d.shape, 1)
    a1 = jnp.min(jnp.where(d == m1, jiota, n), axis=1, keepdims=True)
    d1_ref[0] = m1.T
    i1_ref[0] = a1.T

    m2 = jnp.min(d, axis=0, keepdims=True)
    riota = lax.broadcasted_iota(jnp.int32, d.shape, 0)
    a2 = jnp.min(jnp.where(d == m2, riota, n), axis=0, keepdims=True) + i * _TN
    d2_ref[0] = m2
    i2_ref[0] = a2


def _sc_body(x1x_hbm, x1y_hbm, x1z_hbm, x2x_hbm, x2y_hbm, x2z_hbm,
             d1_hbm, i1_hbm, cv_hbm, ci_hbm,
             x2x_v, x2y_v, x2z_v, x1x_v, x1y_v, x1z_v,
             d1_v, i1_v, cv_v, ci_v):
    c = lax.axis_index("c")
    s = lax.axis_index("s")
    wid = s * 2 + c
    b = wid // _WPB
    w = wid % _WPB
    row0 = _S_TC + w * _RW

    pltpu.sync_copy(x2x_hbm.at[pl.ds(b * _N, _N)], x2x_v)
    pltpu.sync_copy(x2y_hbm.at[pl.ds(b * _N, _N)], x2y_v)
    pltpu.sync_copy(x2z_hbm.at[pl.ds(b * _N, _N)], x2z_v)
    pltpu.sync_copy(x1x_hbm.at[pl.ds(b * _N + row0, _RW)], x1x_v)
    pltpu.sync_copy(x1y_hbm.at[pl.ds(b * _N + row0, _RW)], x1y_v)
    pltpu.sync_copy(x1z_hbm.at[pl.ds(b * _N + row0, _RW)], x1z_v)

    inf16 = jnp.full((16,), jnp.inf, jnp.float32)
    zero16 = jnp.zeros((16,), jnp.int32)
    lane = lax.broadcasted_iota(jnp.int32, (16,), 0)

    def init_body(jc, carry):
        sl = pl.ds(jc * 16, 16)
        cv_v[sl] = inf16
        ci_v[sl] = zero16
        return carry

    lax.fori_loop(0, _N // 16, init_body, 0)

    perms = [jnp.reshape((lane + sh) & 15, (16, 1)) for sh in (8, 4, 2, 1)]

    nr = 8

    def group_body(g, carry):
        base = g * 16
        pxs = x1x_v[pl.ds(base, 16)]
        pys = x1y_v[pl.ds(base, 16)]
        pzs = x1z_v[pl.ds(base, 16)]
        mns = inf16
        mis = zero16
        for r0 in range(0, 16, nr):
            px = [jnp.full((16,), pxs[r0 + q], jnp.float32) for q in range(nr)]
            py = [jnp.full((16,), pys[r0 + q], jnp.float32) for q in range(nr)]
            pz = [jnp.full((16,), pzs[r0 + q], jnp.float32) for q in range(nr)]
            rowbase = jnp.full((16,), row0 + base + r0, jnp.int32)

            def j_body(jc, rcarry):
                rvs, ris = rcarry
                sl = pl.ds(jc * 16, 16)
                bx = x2x_v[sl]
                by = x2y_v[sl]
                bz = x2z_v[sl]
                jidx = lane + jc * 16
                ds = []
                new_rvs, new_ris = [], []
                for q in range(nr):
                    t = bx - px[q]
                    d = t * t
                    t = by - py[q]
                    d = d + t * t
                    t = bz - pz[q]
                    d = d + t * t
                    ds.append(d)
                    m = d < rvs[q]
                    new_rvs.append(jnp.where(m, d, rvs[q]))
                    new_ris.append(jnp.where(m, jidx, ris[q]))
                dm = ds[0]
                qm = jnp.zeros((16,), jnp.int32)
                for q in range(1, nr):
                    mq = ds[q] < dm
                    dm = jnp.where(mq, ds[q], dm)
                    qm = jnp.where(mq, jnp.full((16,), q, jnp.int32), qm)
                cv = cv_v[sl]
                m2 = dm < cv
                cv_v[sl] = jnp.where(m2, dm, cv)
                ci_v[sl] = jnp.where(m2, qm + rowbase, ci_v[sl])
                return tuple(new_rvs), tuple(new_ris)

            init = (tuple(inf16 for _ in range(nr)),
                    tuple(zero16 for _ in range(nr)))
            rvs, ris = lax.fori_loop(0, _N // 16, j_body, init, unroll=4)
            for q in range(nr):
                rv, ri = rvs[q], ris[q]
                for perm in perms:
                    pv = _lane_shuffle(rv, perm)
                    pi = _lane_shuffle(ri, perm)
                    better = (pv < rv) | ((pv == rv) & (pi < ri))
                    rv = jnp.where(better, pv, rv)
                    ri = jnp.where(better, pi, ri)
                msk = lane == (r0 + q)
                mns = jnp.where(msk, rv, mns)
                mis = jnp.where(msk, ri, mis)
        d1_v[pl.ds(base, 16)] = mns
        i1_v[pl.ds(base, 16)] = mis
        return carry

    lax.fori_loop(0, _RW // 16, group_body, 0)

    ns = _N - _S_TC
    pltpu.sync_copy(d1_v, d1_hbm.at[pl.ds(b * ns + w * _RW, _RW)])
    pltpu.sync_copy(i1_v, i1_hbm.at[pl.ds(b * ns + w * _RW, _RW)])
    pltpu.sync_copy(cv_v, cv_hbm.at[pl.ds((b * _WPB + w) * _N, _N)])
    pltpu.sync_copy(ci_v, ci_hbm.at[pl.ds((b * _WPB + w) * _N, _N)])


def _merge_block(d1t_ref, i1t_ref, d1s_ref, i1s_ref,
                 cvt_ref, cit_ref, cvs_ref, cis_ref,
                 d1_ref, i1_ref, d2_ref, i2_ref):
    n = _N
    nb = _S_TC // _TN
    ns = _N - _S_TC
    for bb in range(_B):
        v = cvt_ref[bb * nb]
        ix = cit_ref[bb * nb]
        for k in range(1, nb):
            vk = cvt_ref[bb * nb + k]
            ik = cit_ref[bb * nb + k]
            better = vk < v
            v = jnp.where(better, vk, v)
            ix = jnp.where(better, ik, ix)
        for w in range(_WPB):
            off = (bb * _WPB + w) * n
            vw = cvs_ref[pl.ds(off, n)].reshape(1, n)
            iw = cis_ref[pl.ds(off, n)].reshape(1, n)
            better = vw < v
            v = jnp.where(better, vw, v)
            ix = jnp.where(better, iw, ix)
        d2_ref[bb : bb + 1, :] = v
        i2_ref[bb : bb + 1, :] = ix
        for k in range(nb):
            d1_ref[bb : bb + 1, pl.ds(k * _TN, _TN)] = d1t_ref[bb * nb + k]
            i1_ref[bb : bb + 1, pl.ds(k * _TN, _TN)] = i1t_ref[bb * nb + k]
        d1_ref[bb : bb + 1, pl.ds(_S_TC, ns)] = (
            d1s_ref[pl.ds(bb * ns, ns)].reshape(1, ns))
        i1_ref[bb : bb + 1, pl.ds(_S_TC, ns)] = (
            i1s_ref[pl.ds(bb * ns, ns)].reshape(1, ns))


def kernel(xyz1, xyz2):
    b, n, _ = xyz1.shape
    nb = _S_TC // _TN
    ns = n - _S_TC

    x1x, x1y, x1z, x2x, x2y, x2z = pl.pallas_call(
        _prep_block,
        grid=(b,),
        in_specs=[
            pl.BlockSpec((1, n, 3), lambda bb: (bb, 0, 0)),
            pl.BlockSpec((1, n, 3), lambda bb: (bb, 0, 0)),
        ],
        out_specs=[pl.BlockSpec((n,), lambda bb: (bb,))] * 6,
        out_shape=[jax.ShapeDtypeStruct((b * n,), jnp.float32)] * 6,
    )(xyz1, xyz2)

    sc_call = pl.kernel(
        _sc_body,
        out_type=[
            jax.ShapeDtypeStruct((b * ns,), jnp.float32),
            jax.ShapeDtypeStruct((b * ns,), jnp.int32),
            jax.ShapeDtypeStruct((b * _WPB * n,), jnp.float32),
            jax.ShapeDtypeStruct((b * _WPB * n,), jnp.int32),
        ],
        scratch_types=[
            pltpu.VMEM((n,), jnp.float32),
            pltpu.VMEM((n,), jnp.float32),
            pltpu.VMEM((n,), jnp.float32),
            pltpu.VMEM((_RW,), jnp.float32),
            pltpu.VMEM((_RW,), jnp.float32),
            pltpu.VMEM((_RW,), jnp.float32),
            pltpu.VMEM((_RW,), jnp.float32),
            pltpu.VMEM((_RW,), jnp.int32),
            pltpu.VMEM((n,), jnp.float32),
            pltpu.VMEM((n,), jnp.int32),
        ],
        mesh=plsc.VectorSubcoreMesh(core_axis_name="c", subcore_axis_name="s"),
    )
    d1s, i1s, cvs, cis = sc_call(x1x, x1y, x1z, x2x, x2y, x2z)

    d1t, i1t, cvt, cit = pl.pallas_call(
        _tc_block,
        grid=(b, nb),
        in_specs=[
            pl.BlockSpec((1, _TN, 3), lambda bb, ii: (bb, ii, 0)),
            pl.BlockSpec((n,), lambda bb, ii: (bb,)),
            pl.BlockSpec((n,), lambda bb, ii: (bb,)),
            pl.BlockSpec((n,), lambda bb, ii: (bb,)),
        ],
        out_specs=[
            pl.BlockSpec((1, 1, _TN), lambda bb, ii: (bb * nb + ii, 0, 0)),
            pl.BlockSpec((1, 1, _TN), lambda bb, ii: (bb * nb + ii, 0, 0)),
            pl.BlockSpec((1, 1, n), lambda bb, ii: (bb * nb + ii, 0, 0)),
            pl.BlockSpec((1, 1, n), lambda bb, ii: (bb * nb + ii, 0, 0)),
        ],
        out_shape=[
            jax.ShapeDtypeStruct((b * nb, 1, _TN), jnp.float32),
            jax.ShapeDtypeStruct((b * nb, 1, _TN), jnp.int32),
            jax.ShapeDtypeStruct((b * nb, 1, n), jnp.float32),
            jax.ShapeDtypeStruct((b * nb, 1, n), jnp.int32),
        ],
        compiler_params=pltpu.CompilerParams(
            dimension_semantics=("parallel", "parallel"),
        ),
    )(xyz1, x2x, x2y, x2z)

    d1, i1, d2, i2 = pl.pallas_call(
        _merge_block,
        out_shape=[
            jax.ShapeDtypeStruct((b, n), jnp.float32),
            jax.ShapeDtypeStruct((b, n), jnp.int32),
            jax.ShapeDtypeStruct((b, n), jnp.float32),
            jax.ShapeDtypeStruct((b, n), jnp.int32),
        ],
    )(d1t, i1t, d1s, i1s, cvt, cit, cvs, cis)

    return (d1, d2, i1, i2)


# speedup vs baseline: 1.1363x; 1.1363x over previous
import jax
import jax.numpy as jnp
from jax import lax
from jax.experimental import pallas as pl
from jax.experimental.pallas import tpu as pltpu
from jax.experimental.pallas import tpu_sc as plsc

_N = 4096
_B = 4
_TN = 512
_S_TC = 3072
_NW = 32
_WPB = _NW // _B
_RW = (_N - _S_TC) // _WPB


def _lane_shuffle(v, perm):
    return lax.gather(
        v,
        perm,
        dimension_numbers=lax.GatherDimensionNumbers(
            offset_dims=(), collapsed_slice_dims=(0,), start_index_map=(0,)
        ),
        slice_sizes=(1,),
        mode=lax.GatherScatterMode.PROMISE_IN_BOUNDS,
    )


def _prep_block(x1_ref, x2_ref,
                x1x_ref, x1y_ref, x1z_ref, x2x_ref, x2y_ref, x2z_ref):
    a = x1_ref[0].T
    c = x2_ref[0].T
    x1x_ref[...] = a[0]
    x1y_ref[...] = a[1]
    x1z_ref[...] = a[2]
    x2x_ref[...] = c[0]
    x2y_ref[...] = c[1]
    x2z_ref[...] = c[2]


def _tc_block(x1_ref, x2x_ref, x2y_ref, x2z_ref,
              d1_ref, i1_ref, d2_ref, i2_ref):
    i = pl.program_id(1)
    a = x1_ref[0]
    n = x2x_ref.shape[0]
    bs = (x2x_ref[...], x2y_ref[...], x2z_ref[...])

    d = None
    for k in range(3):
        ak = a[:, k : k + 1]
        bk = bs[k].reshape(1, n)
        t = ak - bk
        t = t * t
        d = t if d is None else d + t

    m1 = jnp.min(d, axis=1, keepdims=True)
    jiota = lax.broadcasted_iota(jnp.int32, d.shape, 1)
    a1 = jnp.min(jnp.where(d == m1, jiota, n), axis=1, keepdims=True)
    d1_ref[0] = m1.T
    i1_ref[0] = a1.T

    m2 = jnp.min(d, axis=0, keepdims=True)
    riota = lax.broadcasted_iota(jnp.int32, d.shape, 0)
    a2 = jnp.min(jnp.where(d == m2, riota, n), axis=0, keepdims=True) + i * _TN

    @pl.when(i == 0)
    def _():
        d2_ref[0] = m2
        i2_ref[0] = a2

    @pl.when(i > 0)
    def _():
        prev_d = d2_ref[0]
        prev_i = i2_ref[0]
        better = m2 < prev_d
        d2_ref[0] = jnp.where(better, m2, prev_d)
        i2_ref[0] = jnp.where(better, a2, prev_i)


def _sc_body(x1x_hbm, x1y_hbm, x1z_hbm, x2x_hbm, x2y_hbm, x2z_hbm,
             d1_hbm, i1_hbm, cv_hbm, ci_hbm,
             x2x_v, x2y_v, x2z_v, x1x_v, x1y_v, x1z_v,
             d1_v, i1_v, cv_v, ci_v):
    c = lax.axis_index("c")
    s = lax.axis_index("s")
    wid = s * 2 + c
    b = wid // _WPB
    w = wid % _WPB
    row0 = _S_TC + w * _RW

    pltpu.sync_copy(x2x_hbm.at[pl.ds(b * _N, _N)], x2x_v)
    pltpu.sync_copy(x2y_hbm.at[pl.ds(b * _N, _N)], x2y_v)
    pltpu.sync_copy(x2z_hbm.at[pl.ds(b * _N, _N)], x2z_v)
    pltpu.sync_copy(x1x_hbm.at[pl.ds(b * _N + row0, _RW)], x1x_v)
    pltpu.sync_copy(x1y_hbm.at[pl.ds(b * _N + row0, _RW)], x1y_v)
    pltpu.sync_copy(x1z_hbm.at[pl.ds(b * _N + row0, _RW)], x1z_v)

    inf16 = jnp.full((16,), jnp.inf, jnp.float32)
    zero16 = jnp.zeros((16,), jnp.int32)
    lane = lax.broadcasted_iota(jnp.int32, (16,), 0)

    def init_body(jc, carry):
        sl = pl.ds(jc * 16, 16)
        cv_v[sl] = inf16
        ci_v[sl] = zero16
        return carry

    lax.fori_loop(0, _N // 16, init_body, 0)

    perms = [jnp.reshape((lane + sh) & 15, (16, 1)) for sh in (8, 4, 2, 1)]

    nr = 8

    def group_body(g, carry):
        base = g * 16
        pxs = x1x_v[pl.ds(base, 16)]
        pys = x1y_v[pl.ds(base, 16)]
        pzs = x1z_v[pl.ds(base, 16)]
        mns = inf16
        mis = zero16
        for r0 in range(0, 16, nr):
            px = [jnp.full((16,), pxs[r0 + q], jnp.float32) for q in range(nr)]
            py = [jnp.full((16,), pys[r0 + q], jnp.float32) for q in range(nr)]
            pz = [jnp.full((16,), pzs[r0 + q], jnp.float32) for q in range(nr)]
            rowbase = jnp.full((16,), row0 + base + r0, jnp.int32)

            def j_body(jc, rcarry):
                rvs, ris = rcarry
                sl = pl.ds(jc * 16, 16)
                bx = x2x_v[sl]
                by = x2y_v[sl]
                bz = x2z_v[sl]
                jidx = lane + jc * 16
                ds = []
                new_rvs, new_ris = [], []
                for q in range(nr):
                    t = bx - px[q]
                    d = t * t
                    t = by - py[q]
                    d = d + t * t
                    t = bz - pz[q]
                    d = d + t * t
                    ds.append(d)
                    m = d < rvs[q]
                    new_rvs.append(jnp.where(m, d, rvs[q]))
                    new_ris.append(jnp.where(m, jidx, ris[q]))
                dm = ds[0]
                qm = jnp.zeros((16,), jnp.int32)
                for q in range(1, nr):
                    mq = ds[q] < dm
                    dm = jnp.where(mq, ds[q], dm)
                    qm = jnp.where(mq, jnp.full((16,), q, jnp.int32), qm)
                cv = cv_v[sl]
                m2 = dm < cv
                cv_v[sl] = jnp.where(m2, dm, cv)
                ci_v[sl] = jnp.where(m2, qm + rowbase, ci_v[sl])
                return tuple(new_rvs), tuple(new_ris)

            init = (tuple(inf16 for _ in range(nr)),
                    tuple(zero16 for _ in range(nr)))
            rvs, ris = lax.fori_loop(0, _N // 16, j_body, init, unroll=4)
            for q in range(nr):
                rv, ri = rvs[q], ris[q]
                for perm in perms:
                    pv = _lane_shuffle(rv, perm)
                    pi = _lane_shuffle(ri, perm)
                    better = (pv < rv) | ((pv == rv) & (pi < ri))
                    rv = jnp.where(better, pv, rv)
                    ri = jnp.where(better, pi, ri)
                msk = lane == (r0 + q)
                mns = jnp.where(msk, rv, mns)
                mis = jnp.where(msk, ri, mis)
        d1_v[pl.ds(base, 16)] = mns
        i1_v[pl.ds(base, 16)] = mis
        return carry

    lax.fori_loop(0, _RW // 16, group_body, 0)

    ns = _N - _S_TC
    pltpu.sync_copy(d1_v, d1_hbm.at[pl.ds(b * ns + w * _RW, _RW)])
    pltpu.sync_copy(i1_v, i1_hbm.at[pl.ds(b * ns + w * _RW, _RW)])
    pltpu.sync_copy(cv_v, cv_hbm.at[pl.ds((b * _WPB + w) * _N, _N)])
    pltpu.sync_copy(ci_v, ci_hbm.at[pl.ds((b * _WPB + w) * _N, _N)])


def _merge_block(d1t_ref, i1t_ref, d1s_ref, i1s_ref,
                 cvt_ref, cit_ref, cvs_ref, cis_ref,
                 d1_ref, i1_ref, d2_ref, i2_ref):
    n = _N
    nb = _S_TC // _TN
    ns = _N - _S_TC
    for bb in range(_B):
        v = cvt_ref[bb]
        ix = cit_ref[bb]
        for w in range(_WPB):
            off = (bb * _WPB + w) * n
            vw = cvs_ref[pl.ds(off, n)].reshape(1, n)
            iw = cis_ref[pl.ds(off, n)].reshape(1, n)
            better = vw < v
            v = jnp.where(better, vw, v)
            ix = jnp.where(better, iw, ix)
        d2_ref[bb : bb + 1, :] = v
        i2_ref[bb : bb + 1, :] = ix
        for k in range(nb):
            d1_ref[bb : bb + 1, pl.ds(k * _TN, _TN)] = d1t_ref[bb * nb + k]
            i1_ref[bb : bb + 1, pl.ds(k * _TN, _TN)] = i1t_ref[bb * nb + k]
        d1_ref[bb : bb + 1, pl.ds(_S_TC, ns)] = (
            d1s_ref[pl.ds(bb * ns, ns)].reshape(1, ns))
        i1_ref[bb : bb + 1, pl.ds(_S_TC, ns)] = (
            i1s_ref[pl.ds(bb * ns, ns)].reshape(1, ns))


def kernel(xyz1, xyz2):
    b, n, _ = xyz1.shape
    nb = _S_TC // _TN
    ns = n - _S_TC

    x1x, x1y, x1z, x2x, x2y, x2z = pl.pallas_call(
        _prep_block,
        grid=(b,),
        in_specs=[
            pl.BlockSpec((1, n, 3), lambda bb: (bb, 0, 0)),
            pl.BlockSpec((1, n, 3), lambda bb: (bb, 0, 0)),
        ],
        out_specs=[pl.BlockSpec((n,), lambda bb: (bb,))] * 6,
        out_shape=[jax.ShapeDtypeStruct((b * n,), jnp.float32)] * 6,
    )(xyz1, xyz2)

    sc_call = pl.kernel(
        _sc_body,
        out_type=[
            jax.ShapeDtypeStruct((b * ns,), jnp.float32),
            jax.ShapeDtypeStruct((b * ns,), jnp.int32),
            jax.ShapeDtypeStruct((b * _WPB * n,), jnp.float32),
            jax.ShapeDtypeStruct((b * _WPB * n,), jnp.int32),
        ],
        scratch_types=[
            pltpu.VMEM((n,), jnp.float32),
            pltpu.VMEM((n,), jnp.float32),
            pltpu.VMEM((n,), jnp.float32),
            pltpu.VMEM((_RW,), jnp.float32),
            pltpu.VMEM((_RW,), jnp.float32),
            pltpu.VMEM((_RW,), jnp.float32),
            pltpu.VMEM((_RW,), jnp.float32),
            pltpu.VMEM((_RW,), jnp.int32),
            pltpu.VMEM((n,), jnp.float32),
            pltpu.VMEM((n,), jnp.int32),
        ],
        mesh=plsc.VectorSubcoreMesh(core_axis_name="c", subcore_axis_name="s"),
    )
    d1s, i1s, cvs, cis = sc_call(x1x, x1y, x1z, x2x, x2y, x2z)

    d1t, i1t, cvt, cit = pl.pallas_call(
        _tc_block,
        grid=(b, nb),
        in_specs=[
            pl.BlockSpec((1, _TN, 3), lambda bb, ii: (bb, ii, 0)),
            pl.BlockSpec((n,), lambda bb, ii: (bb,)),
            pl.BlockSpec((n,), lambda bb, ii: (bb,)),
            pl.BlockSpec((n,), lambda bb, ii: (bb,)),
        ],
        out_specs=[
            pl.BlockSpec((1, 1, _TN), lambda bb, ii: (bb * nb + ii, 0, 0)),
            pl.BlockSpec((1, 1, _TN), lambda bb, ii: (bb * nb + ii, 0, 0)),
            pl.BlockSpec((1, 1, n), lambda bb, ii: (bb, 0, 0)),
            pl.BlockSpec((1, 1, n), lambda bb, ii: (bb, 0, 0)),
        ],
        out_shape=[
            jax.ShapeDtypeStruct((b * nb, 1, _TN), jnp.float32),
            jax.ShapeDtypeStruct((b * nb, 1, _TN), jnp.int32),
            jax.ShapeDtypeStruct((b, 1, n), jnp.float32),
            jax.ShapeDtypeStruct((b, 1, n), jnp.int32),
        ],
        compiler_params=pltpu.CompilerParams(
            dimension_semantics=("parallel", "arbitrary"),
        ),
    )(xyz1, x2x, x2y, x2z)

    d1, i1, d2, i2 = pl.pallas_call(
        _merge_block,
        out_shape=[
            jax.ShapeDtypeStruct((b, n), jnp.float32),
            jax.ShapeDtypeStruct((b, n), jnp.int32),
            jax.ShapeDtypeStruct((b, n), jnp.float32),
            jax.ShapeDtypeStruct((b, n), jnp.int32),
        ],
    )(d1t, i1t, d1s, i1s, cvt, cit, cvs, cis)

    return (d1, d2, i1, i2)


# speedup vs baseline: 1.1505x; 1.0126x over previous
import jax
import jax.numpy as jnp
from jax import lax
from jax.experimental import pallas as pl
from jax.experimental.pallas import tpu as pltpu
from jax.experimental.pallas import tpu_sc as plsc

_N = 4096
_B = 4
_TN = 1024
_S_TC = 3072
_NW = 32
_WPB = _NW // _B
_RW = (_N - _S_TC) // _WPB


def _lane_shuffle(v, perm):
    return lax.gather(
        v,
        perm,
        dimension_numbers=lax.GatherDimensionNumbers(
            offset_dims=(), collapsed_slice_dims=(0,), start_index_map=(0,)
        ),
        slice_sizes=(1,),
        mode=lax.GatherScatterMode.PROMISE_IN_BOUNDS,
    )


def _prep_block(x1_ref, x2_ref,
                x1x_ref, x1y_ref, x1z_ref, x2x_ref, x2y_ref, x2z_ref):
    a = x1_ref[0].T
    c = x2_ref[0].T
    x1x_ref[...] = a[0]
    x1y_ref[...] = a[1]
    x1z_ref[...] = a[2]
    x2x_ref[...] = c[0]
    x2y_ref[...] = c[1]
    x2z_ref[...] = c[2]


def _tc_block(x1_ref, x2x_ref, x2y_ref, x2z_ref,
              d1_ref, i1_ref, d2_ref, i2_ref):
    i = pl.program_id(1)
    a = x1_ref[0]
    n = x2x_ref.shape[0]
    bs = (x2x_ref[...], x2y_ref[...], x2z_ref[...])

    d = None
    for k in range(3):
        ak = a[:, k : k + 1]
        bk = bs[k].reshape(1, n)
        t = ak - bk
        t = t * t
        d = t if d is None else d + t

    m1 = jnp.min(d, axis=1, keepdims=True)
    jiota = lax.broadcasted_iota(jnp.int32, d.shape, 1)
    a1 = jnp.min(jnp.where(d == m1, jiota, n), axis=1, keepdims=True)
    d1_ref[0] = m1.T
    i1_ref[0] = a1.T

    m2 = jnp.min(d, axis=0, keepdims=True)
    riota = lax.broadcasted_iota(jnp.int32, d.shape, 0)
    a2 = jnp.min(jnp.where(d == m2, riota, n), axis=0, keepdims=True) + i * _TN

    @pl.when(i == 0)
    def _():
        d2_ref[0] = m2
        i2_ref[0] = a2

    @pl.when(i > 0)
    def _():
        prev_d = d2_ref[0]
        prev_i = i2_ref[0]
        better = m2 < prev_d
        d2_ref[0] = jnp.where(better, m2, prev_d)
        i2_ref[0] = jnp.where(better, a2, prev_i)


def _sc_body(x1x_hbm, x1y_hbm, x1z_hbm, x2x_hbm, x2y_hbm, x2z_hbm,
             d1_hbm, i1_hbm, cv_hbm, ci_hbm,
             x2x_v, x2y_v, x2z_v, x1x_v, x1y_v, x1z_v,
             d1_v, i1_v, cv_v, ci_v):
    c = lax.axis_index("c")
    s = lax.axis_index("s")
    wid = s * 2 + c
    b = wid // _WPB
    w = wid % _WPB
    row0 = _S_TC + w * _RW

    pltpu.sync_copy(x2x_hbm.at[pl.ds(b * _N, _N)], x2x_v)
    pltpu.sync_copy(x2y_hbm.at[pl.ds(b * _N, _N)], x2y_v)
    pltpu.sync_copy(x2z_hbm.at[pl.ds(b * _N, _N)], x2z_v)
    pltpu.sync_copy(x1x_hbm.at[pl.ds(b * _N + row0, _RW)], x1x_v)
    pltpu.sync_copy(x1y_hbm.at[pl.ds(b * _N + row0, _RW)], x1y_v)
    pltpu.sync_copy(x1z_hbm.at[pl.ds(b * _N + row0, _RW)], x1z_v)

    inf16 = jnp.full((16,), jnp.inf, jnp.float32)
    zero16 = jnp.zeros((16,), jnp.int32)
    lane = lax.broadcasted_iota(jnp.int32, (16,), 0)

    def init_body(jc, carry):
        sl = pl.ds(jc * 16, 16)
        cv_v[sl] = inf16
        ci_v[sl] = zero16
        return carry

    lax.fori_loop(0, _N // 16, init_body, 0)

    perms = [jnp.reshape((lane + sh) & 15, (16, 1)) for sh in (8, 4, 2, 1)]

    nr = 8

    def group_body(g, carry):
        base = g * 16
        pxs = x1x_v[pl.ds(base, 16)]
        pys = x1y_v[pl.ds(base, 16)]
        pzs = x1z_v[pl.ds(base, 16)]
        mns = inf16
        mis = zero16
        for r0 in range(0, 16, nr):
            px = [jnp.full((16,), pxs[r0 + q], jnp.float32) for q in range(nr)]
            py = [jnp.full((16,), pys[r0 + q], jnp.float32) for q in range(nr)]
            pz = [jnp.full((16,), pzs[r0 + q], jnp.float32) for q in range(nr)]
            rowbase = jnp.full((16,), row0 + base + r0, jnp.int32)

            def j_body(jc, rcarry):
                rvs, ris = rcarry
                sl = pl.ds(jc * 16, 16)
                bx = x2x_v[sl]
                by = x2y_v[sl]
                bz = x2z_v[sl]
                jidx = lane + jc * 16
                ds = []
                new_rvs, new_ris = [], []
                for q in range(nr):
                    t = bx - px[q]
                    d = t * t
                    t = by - py[q]
                    d = d + t * t
                    t = bz - pz[q]
                    d = d + t * t
                    ds.append(d)
                    m = d < rvs[q]
                    new_rvs.append(jnp.where(m, d, rvs[q]))
                    new_ris.append(jnp.where(m, jidx, ris[q]))
                dm = ds[0]
                qm = jnp.zeros((16,), jnp.int32)
                for q in range(1, nr):
                    mq = ds[q] < dm
                    dm = jnp.where(mq, ds[q], dm)
                    qm = jnp.where(mq, jnp.full((16,), q, jnp.int32), qm)
                cv = cv_v[sl]
                m2 = dm < cv
                cv_v[sl] = jnp.where(m2, dm, cv)
                ci_v[sl] = jnp.where(m2, qm + rowbase, ci_v[sl])
                return tuple(new_rvs), tuple(new_ris)

            init = (tuple(inf16 for _ in range(nr)),
                    tuple(zero16 for _ in range(nr)))
            rvs, ris = lax.fori_loop(0, _N // 16, j_body, init, unroll=4)
            for q in range(nr):
                rv, ri = rvs[q], ris[q]
                for perm in perms:
                    pv = _lane_shuffle(rv, perm)
                    pi = _lane_shuffle(ri, perm)
                    better = (pv < rv) | ((pv == rv) & (pi < ri))
                    rv = jnp.where(better, pv, rv)
                    ri = jnp.where(better, pi, ri)
                msk = lane == (r0 + q)
                mns = jnp.where(msk, rv, mns)
                mis = jnp.where(msk, ri, mis)
        d1_v[pl.ds(base, 16)] = mns
        i1_v[pl.ds(base, 16)] = mis
        return carry

    lax.fori_loop(0, _RW // 16, group_body, 0)

    ns = _N - _S_TC
    pltpu.sync_copy(d1_v, d1_hbm.at[pl.ds(b * ns + w * _RW, _RW)])
    pltpu.sync_copy(i1_v, i1_hbm.at[pl.ds(b * ns + w * _RW, _RW)])
    pltpu.sync_copy(cv_v, cv_hbm.at[pl.ds((b * _WPB + w) * _N, _N)])
    pltpu.sync_copy(ci_v, ci_hbm.at[pl.ds((b * _WPB + w) * _N, _N)])


def _merge_block(d1t_ref, i1t_ref, d1s_ref, i1s_ref,
                 cvt_ref, cit_ref, cvs_ref, cis_ref,
                 d1_ref, i1_ref, d2_ref, i2_ref):
    n = _N
    nb = _S_TC // _TN
    ns = _N - _S_TC
    for bb in range(_B):
        v = cvt_ref[bb]
        ix = cit_ref[bb]
        for w in range(_WPB):
            off = (bb * _WPB + w) * n
            vw = cvs_ref[pl.ds(off, n)].reshape(1, n)
            iw = cis_ref[pl.ds(off, n)].reshape(1, n)
            better = vw < v
            v = jnp.where(better, vw, v)
            ix = jnp.where(better, iw, ix)
        d2_ref[bb : bb + 1, :] = v
        i2_ref[bb : bb + 1, :] = ix
        for k in range(nb):
            d1_ref[bb : bb + 1, pl.ds(k * _TN, _TN)] = d1t_ref[bb * nb + k]
            i1_ref[bb : bb + 1, pl.ds(k * _TN, _TN)] = i1t_ref[bb * nb + k]
        d1_ref[bb : bb + 1, pl.ds(_S_TC, ns)] = (
            d1s_ref[pl.ds(bb * ns, ns)].reshape(1, ns))
        i1_ref[bb : bb + 1, pl.ds(_S_TC, ns)] = (
            i1s_ref[pl.ds(bb * ns, ns)].reshape(1, ns))


def kernel(xyz1, xyz2):
    b, n, _ = xyz1.shape
    nb = _S_TC // _TN
    ns = n - _S_TC

    x1x, x1y, x1z, x2x, x2y, x2z = pl.pallas_call(
        _prep_block,
        grid=(b,),
        in_specs=[
            pl.BlockSpec((1, n, 3), lambda bb: (bb, 0, 0)),
            pl.BlockSpec((1, n, 3), lambda bb: (bb, 0, 0)),
        ],
        out_specs=[pl.BlockSpec((n,), lambda bb: (bb,))] * 6,
        out_shape=[jax.ShapeDtypeStruct((b * n,), jnp.float32)] * 6,
    )(xyz1, xyz2)

    sc_call = pl.kernel(
        _sc_body,
        out_type=[
            jax.ShapeDtypeStruct((b * ns,), jnp.float32),
            jax.ShapeDtypeStruct((b * ns,), jnp.int32),
            jax.ShapeDtypeStruct((b * _WPB * n,), jnp.float32),
            jax.ShapeDtypeStruct((b * _WPB * n,), jnp.int32),
        ],
        scratch_types=[
            pltpu.VMEM((n,), jnp.float32),
            pltpu.VMEM((n,), jnp.float32),
            pltpu.VMEM((n,), jnp.float32),
            pltpu.VMEM((_RW,), jnp.float32),
            pltpu.VMEM((_RW,), jnp.float32),
            pltpu.VMEM((_RW,), jnp.float32),
            pltpu.VMEM((_RW,), jnp.float32),
            pltpu.VMEM((_RW,), jnp.int32),
            pltpu.VMEM((n,), jnp.float32),
            pltpu.VMEM((n,), jnp.int32),
        ],
        mesh=plsc.VectorSubcoreMesh(core_axis_name="c", subcore_axis_name="s"),
    )
    d1s, i1s, cvs, cis = sc_call(x1x, x1y, x1z, x2x, x2y, x2z)

    d1t, i1t, cvt, cit = pl.pallas_call(
        _tc_block,
        grid=(b, nb),
        in_specs=[
            pl.BlockSpec((1, _TN, 3), lambda bb, ii: (bb, ii, 0)),
            pl.BlockSpec((n,), lambda bb, ii: (bb,)),
            pl.BlockSpec((n,), lambda bb, ii: (bb,)),
            pl.BlockSpec((n,), lambda bb, ii: (bb,)),
        ],
        out_specs=[
            pl.BlockSpec((1, 1, _TN), lambda bb, ii: (bb * nb + ii, 0, 0)),
            pl.BlockSpec((1, 1, _TN), lambda bb, ii: (bb * nb + ii, 0, 0)),
            pl.BlockSpec((1, 1, n), lambda bb, ii: (bb, 0, 0)),
            pl.BlockSpec((1, 1, n), lambda bb, ii: (bb, 0, 0)),
        ],
        out_shape=[
            jax.ShapeDtypeStruct((b * nb, 1, _TN), jnp.float32),
            jax.ShapeDtypeStruct((b * nb, 1, _TN), jnp.int32),
            jax.ShapeDtypeStruct((b, 1, n), jnp.float32),
            jax.ShapeDtypeStruct((b, 1, n), jnp.int32),
        ],
        compiler_params=pltpu.CompilerParams(
            dimension_semantics=("parallel", "arbitrary"),
        ),
    )(xyz1, x2x, x2y, x2z)

    d1, i1, d2, i2 = pl.pallas_call(
        _merge_block,
        out_shape=[
            jax.ShapeDtypeStruct((b, n), jnp.float32),
            jax.ShapeDtypeStruct((b, n), jnp.int32),
            jax.ShapeDtypeStruct((b, n), jnp.float32),
            jax.ShapeDtypeStruct((b, n), jnp.int32),
        ],
    )(d1t, i1t, d1s, i1s, cvt, cit, cvs, cis)

    return (d1, d2, i1, i2)
